# Initial kernel scaffold; baseline (speedup 1.0000x reference)
import jax
import jax.numpy as jnp
from jax.experimental import pallas as pl


def kernel(input_0):
    raise NotImplementedError("write your pallas kernel here")


# speedup vs baseline: 1.0786x; 1.0786x over previous
import jax
import jax.numpy as jnp
from jax.experimental import pallas as pl
from jax.experimental.pallas import tpu as pltpu

_ROWS = 256
_COLS = 128


def _cumsum_body(x_ref, o_ref):
    x = x_ref[:]

    ii = jax.lax.broadcasted_iota(jnp.int32, (_COLS, _COLS), 0)
    jj = jax.lax.broadcasted_iota(jnp.int32, (_COLS, _COLS), 1)
    upper = (ii <= jj).astype(jnp.float32)

    c = jax.lax.dot(x, upper, preferred_element_type=jnp.float32,
                    precision=jax.lax.Precision.HIGHEST)

    rr = jax.lax.broadcasted_iota(jnp.int32, (_ROWS, _ROWS), 0)
    cc = jax.lax.broadcasted_iota(jnp.int32, (_ROWS, _ROWS), 1)
    strict_lower = (rr > cc).astype(jnp.float32)
    row_tot = c[:, _COLS - 1:_COLS]
    prefix = jax.lax.dot(strict_lower, row_tot,
                         preferred_element_type=jnp.float32,
                         precision=jax.lax.Precision.HIGHEST)

    o_ref[:] = c + prefix


def kernel(input_0):
    x = input_0.reshape(_ROWS, _COLS)
    out = pl.pallas_call(
        _cumsum_body,
        out_shape=jax.ShapeDtypeStruct((_ROWS, _COLS), jnp.float32),
        in_specs=[pl.BlockSpec((_ROWS, _COLS), lambda: (0, 0))],
        out_specs=pl.BlockSpec((_ROWS, _COLS), lambda: (0, 0)),
    )(x)
    return out.reshape(32768)


# speedup vs baseline: 1.2021x; 1.1145x over previous
import jax
import jax.numpy as jnp
from jax.experimental import pallas as pl
from jax.experimental.pallas import tpu as pltpu

_ROWS = 256
_COLS = 128


def _cumsum_body(x_ref, o_ref):
    x = x_ref[:]

    ii = jax.lax.broadcasted_iota(jnp.int32, (_COLS, _COLS), 0)
    jj = jax.lax.broadcasted_iota(jnp.int32, (_COLS, _COLS), 1)
    upper = (ii <= jj).astype(jnp.float32)

    c = jax.lax.dot(x, upper, preferred_element_type=jnp.float32)

    rr = jax.lax.broadcasted_iota(jnp.int32, (_ROWS, _ROWS), 0)
    cc = jax.lax.broadcasted_iota(jnp.int32, (_ROWS, _ROWS), 1)
    strict_lower = (rr > cc).astype(jnp.float32)
    row_tot = c[:, _COLS - 1:_COLS]
    prefix = jax.lax.dot(strict_lower, row_tot,
                         preferred_element_type=jnp.float32,
                         precision=jax.lax.Precision.HIGHEST)

    o_ref[:] = c + prefix


def kernel(input_0):
    x = input_0.reshape(_ROWS, _COLS)
    out = pl.pallas_call(
        _cumsum_body,
        out_shape=jax.ShapeDtypeStruct((_ROWS, _COLS), jnp.float32),
        in_specs=[pl.BlockSpec((_ROWS, _COLS), lambda: (0, 0))],
        out_specs=pl.BlockSpec((_ROWS, _COLS), lambda: (0, 0)),
    )(x)
    return out.reshape(32768)


# speedup vs baseline: 1.2707x; 1.0570x over previous
import jax
import jax.numpy as jnp
from jax.experimental import pallas as pl
from jax.experimental.pallas import tpu as pltpu

_ROWS = 256
_COLS = 128


def _cumsum_body(x_ref, o_ref):
    x = x_ref[:]

    ii = jax.lax.broadcasted_iota(jnp.int32, (_COLS, _COLS), 0)
    jj = jax.lax.broadcasted_iota(jnp.int32, (_COLS, _COLS), 1)
    upper = (ii <= jj).astype(jnp.float32)

    c = jax.lax.dot(x, upper, preferred_element_type=jnp.float32)

    rr = jax.lax.broadcasted_iota(jnp.int32, (_ROWS, _ROWS), 0)
    cc = jax.lax.broadcasted_iota(jnp.int32, (_ROWS, _ROWS), 1)
    strict_lower = (rr > cc).astype(jnp.float32)
    row_tot = c[:, _COLS - 1:_COLS]
    hi = row_tot.astype(jnp.bfloat16).astype(jnp.float32)
    lo = row_tot - hi
    both = jnp.concatenate([hi, lo], axis=1)
    pp = jax.lax.dot(strict_lower, both,
                     preferred_element_type=jnp.float32)
    prefix = pp[:, 0:1] + pp[:, 1:2]

    o_ref[:] = c + prefix


def kernel(input_0):
    x = input_0.reshape(_ROWS, _COLS)
    out = pl.pallas_call(
        _cumsum_body,
        out_shape=jax.ShapeDtypeStruct((_ROWS, _COLS), jnp.float32),
        in_specs=[pl.BlockSpec((_ROWS, _COLS), lambda: (0, 0))],
        out_specs=pl.BlockSpec((_ROWS, _COLS), lambda: (0, 0)),
    )(x)
    return out.reshape(32768)
